# baseline (speedup 1.0000x reference)
import jax
import jax.numpy as jnp
from jax import lax
from jax.experimental import pallas as pl
from jax.experimental.pallas import tpu as pltpu
from jax.experimental.pallas import tpu_sc as plsc

E = 4
V = 1_000_000
D = 16
NW = 32
B = 16384
F = 26
N = B * F
WPE = NW // E
PER_W = N // WPE
BPW = B // WPE
CB = 64
G = CB * F
NCH = BPW // CB


def _sc_body(idx_hbm, emb_hbm, out_hbm, idx_v, buf0, buf1, sg0, sg1, so0, so1):
    wid = lax.axis_index("s") * 2 + lax.axis_index("c")
    e = wid // WPE
    slot = wid % WPE
    base = slot * PER_W
    b0 = slot * BPW
    pltpu.sync_copy(idx_hbm.at[pl.ds(base, PER_W)], idx_v)

    bufs = (buf0, buf1)
    g_sems = (sg0, sg1)
    o_sems = (so0, so1)

    def gather(k):
        return pltpu.async_copy(
            emb_hbm.at[e].at[idx_v.at[pl.ds(k * G, G)]],
            bufs[k % 2],
            g_sems[k % 2],
        )

    def drain_out(k):
        pltpu.make_async_copy(
            emb_hbm.at[e].at[pl.ds(0, G)], bufs[k % 2], o_sems[k % 2]
        ).wait()

    def emit_out(k):
        buf, sem = bufs[k % 2], o_sems[k % 2]
        bb = b0 + k * CB

        def body(j, _):
            pltpu.async_copy(
                buf.at[pl.ds(j * F, F)], out_hbm.at[e, bb + j], sem
            )
            return 0

        lax.fori_loop(0, CB, body, 0)

    gather(0)
    for k in range(NCH):
        if k + 1 < NCH:
            if k >= 1:
                drain_out(k - 1)
            gather(k + 1)
        pltpu.make_async_copy(
            emb_hbm.at[e].at[idx_v.at[pl.ds(k * G, G)]],
            bufs[k % 2],
            g_sems[k % 2],
        ).wait()
        emit_out(k)
    drain_out(NCH - 2)
    drain_out(NCH - 1)


def _lookup(idx_flat, embedding):
    mesh = plsc.VectorSubcoreMesh(core_axis_name="c", subcore_axis_name="s")
    return pl.kernel(
        _sc_body,
        out_type=jax.ShapeDtypeStruct((E, B, F, D), jnp.float32),
        mesh=mesh,
        scratch_types=[
            pltpu.VMEM((PER_W,), jnp.int32),
            pltpu.VMEM((G, D), jnp.float32),
            pltpu.VMEM((G, D), jnp.float32),
            pltpu.SemaphoreType.DMA,
            pltpu.SemaphoreType.DMA,
            pltpu.SemaphoreType.DMA,
            pltpu.SemaphoreType.DMA,
        ],
        compiler_params=pltpu.CompilerParams(use_tc_tiling_on_sc=False),
    )(idx_flat, embedding)


def kernel(indices, embedding):
    return _lookup(indices.reshape(-1), embedding)
